# Initial kernel scaffold; baseline (speedup 1.0000x reference)
import functools

import jax
import jax.numpy as jnp
from jax import lax
from jax.experimental import pallas as pl
from jax.experimental.pallas import tpu as pltpu
from jax.experimental.pallas import tpu_sc as plsc

N = 10000
E = 160000
D_IN = 256
D_H = 512
D_OUT = 3
G = 64
W = 16
NC, NS = 2, 16
NW = NC * NS
EPW = E // NW
C = 125
K = EPW // C
RPS = N // NS
RBLK = 1000


def _mlp_body(x_ref, w1_ref, b1_ref, w2_ref, o_ref):
    h = jnp.dot(x_ref[...], w1_ref[...], preferred_element_type=jnp.float32)
    h = jnp.maximum(h + b1_ref[...], 0.0)
    o_ref[...] = jnp.dot(h, w2_ref[...], preferred_element_type=jnp.float32)


def _mlp(x, W1, b1r, w2p):
    return pl.pallas_call(
        _mlp_body,
        grid=(N // RBLK,),
        in_specs=[
            pl.BlockSpec((RBLK, D_IN), lambda i: (i, 0)),
            pl.BlockSpec((D_IN, D_H), lambda i: (0, 0)),
            pl.BlockSpec((1, D_H), lambda i: (0, 0)),
            pl.BlockSpec((D_H, W), lambda i: (0, 0)),
        ],
        out_specs=pl.BlockSpec((RBLK, W), lambda i: (i, 0)),
        out_shape=jax.ShapeDtypeStruct((N, W), jnp.float32),
    )(x, W1, b1r, w2p)


def _sc_scatter(hw2, src, dst, zeros):
    mesh = plsc.VectorSubcoreMesh(core_axis_name="c", subcore_axis_name="s")

    @functools.partial(
        pl.kernel,
        out_type=jax.ShapeDtypeStruct((NC, N, W), jnp.float32),
        mesh=mesh,
        scratch_types=[
            pltpu.VMEM((K, C), jnp.int32),
            pltpu.VMEM((K, C), jnp.int32),
            pltpu.VMEM((C, W), jnp.float32),
            pltpu.VMEM_SHARED((N, W), jnp.float32),
            pltpu.SemaphoreType.DMA,
        ],
    )
    def body(hw2_hbm, src_hbm, dst_hbm, zeros_hbm, out_hbm,
             src_v, dst_v, rows_v, agg, sem):
        c = lax.axis_index("c")
        s = lax.axis_index("s")
        wid = s * NC + c
        pltpu.sync_copy(zeros_hbm.at[pl.ds(s * RPS, RPS)],
                        agg.at[pl.ds(s * RPS, RPS)])
        pltpu.sync_copy(src_hbm.at[wid], src_v)
        pltpu.sync_copy(dst_hbm.at[wid], dst_v)
        plsc.subcore_barrier()

        def chunk(j, carry):
            pltpu.async_copy(hw2_hbm.at[src_v.at[j]], rows_v, sem).wait()
            pltpu.sync_copy(rows_v, agg.at[dst_v.at[j]], add=True)
            return carry

        lax.fori_loop(0, K, chunk, 0)
        plsc.subcore_barrier()
        pltpu.sync_copy(agg.at[pl.ds(s * RPS, RPS)],
                        out_hbm.at[c, pl.ds(s * RPS, RPS)])

    return body(hw2, src, dst, zeros)


def _finish_body(hw2_ref, agg_ref, batch_ref, b2_ref, eps_ref, t_ref, o_ref):
    eps = eps_ref[0, 0]
    invt = 1.0 / t_ref[0, 0]
    z = (1.0 + eps) * hw2_ref[...] + agg_ref[0] + agg_ref[1] + b2_ref[...]
    col = lax.broadcasted_iota(jnp.int32, (N, W), 1)
    valid = col < D_OUT
    m = jnp.max(jnp.where(valid, z, -jnp.inf), axis=1, keepdims=True)
    e = jnp.where(valid, jnp.exp(z - m), 0.0)
    ls = z - m - jnp.log(jnp.sum(e, axis=1, keepdims=True))
    lsx = jnp.where(valid, ls, jnp.where(col == D_OUT, 1.0, 0.0))
    onehot = (batch_ref[...] == lax.broadcasted_iota(jnp.int32, (N, G), 1))
    sums = lax.dot_general(onehot.astype(jnp.float32), lsx,
                           (((0,), (0,)), ((), ())),
                           preferred_element_type=jnp.float32)
    colg = lax.broadcasted_iota(jnp.int32, (G, W), 1)
    cnt = jnp.sum(jnp.where(colg == D_OUT, sums, 0.0), axis=1, keepdims=True)
    pooled = sums / jnp.maximum(cnt, 1.0) * invt
    validg = colg < D_OUT
    pm = jnp.max(jnp.where(validg, pooled, -jnp.inf), axis=1, keepdims=True)
    pe = jnp.where(validg, jnp.exp(pooled - pm), 0.0)
    o_ref[...] = pooled - pm - jnp.log(jnp.sum(pe, axis=1, keepdims=True))


def _finish(hw2, aggp, batch2d, b2p, eps, T):
    return pl.pallas_call(
        _finish_body,
        in_specs=[
            pl.BlockSpec((N, W), lambda: (0, 0)),
            pl.BlockSpec((NC, N, W), lambda: (0, 0, 0)),
            pl.BlockSpec((N, 1), lambda: (0, 0)),
            pl.BlockSpec((1, W), lambda: (0, 0)),
            pl.BlockSpec(memory_space=pltpu.SMEM),
            pl.BlockSpec(memory_space=pltpu.SMEM),
        ],
        out_specs=pl.BlockSpec((G, W), lambda: (0, 0)),
        out_shape=jax.ShapeDtypeStruct((G, W), jnp.float32),
    )(hw2, aggp, batch2d, b2p, eps, T)


def kernel(x, edge_index, batch, W1, b1, W2, b2, eps, T):
    w2p = jnp.pad(W2, ((0, 0), (0, W - D_OUT)))
    b2p = jnp.pad(b2, (0, W - D_OUT)).reshape(1, W)
    b1r = b1.reshape(1, D_H)
    hw2 = _mlp(x, W1, b1r, w2p)
    src = edge_index[0].reshape(NW, K, C)
    dst = edge_index[1].reshape(NW, K, C)
    zeros = jnp.zeros((N, W), jnp.float32)
    aggp = _sc_scatter(hw2, src, dst, zeros)
    out16 = _finish(hw2, aggp, batch.reshape(N, 1), b2p,
                    eps.reshape(1, 1), T.reshape(1, 1))
    return out16[:, :D_OUT]


# speedup vs baseline: 27.8392x; 27.8392x over previous
import functools

import jax
import jax.numpy as jnp
from jax import lax
from jax.experimental import pallas as pl
from jax.experimental.pallas import tpu as pltpu
from jax.experimental.pallas import tpu_sc as plsc

N = 10000
E = 160000
D_IN = 256
D_H = 512
D_OUT = 3
G = 64
W = 8
NC, NS = 2, 16
NW = NC * NS
C = 128
K = 40
E_PAD = NW * K * C
N_PAD = 10240
RPS = N_PAD // NS
RBLK = 1024
NB = 8
GP = 4
KG = K // GP


def _mlp_body(x_ref, w1_ref, b1_ref, w2_ref, o_ref):
    h = jnp.dot(x_ref[...], w1_ref[...], preferred_element_type=jnp.float32)
    h = jnp.maximum(h + b1_ref[...], 0.0)
    o_ref[...] = jnp.dot(h, w2_ref[...], preferred_element_type=jnp.float32)


def _mlp(x, W1, b1r, w2p):
    return pl.pallas_call(
        _mlp_body,
        grid=(N_PAD // RBLK,),
        in_specs=[
            pl.BlockSpec((RBLK, D_IN), lambda i: (i, 0)),
            pl.BlockSpec((D_IN, D_H), lambda i: (0, 0)),
            pl.BlockSpec((1, D_H), lambda i: (0, 0)),
            pl.BlockSpec((D_H, W), lambda i: (0, 0)),
        ],
        out_specs=pl.BlockSpec((RBLK, W), lambda i: (i, 0)),
        out_shape=jax.ShapeDtypeStruct((N_PAD, W), jnp.float32),
    )(x, W1, b1r, w2p)


def _sc_scatter(hw2, ei, zeros):
    mesh = plsc.VectorSubcoreMesh(core_axis_name="c", subcore_axis_name="s")

    @functools.partial(
        pl.kernel,
        out_type=jax.ShapeDtypeStruct((NC, N_PAD, W), jnp.float32),
        mesh=mesh,
        compiler_params=pltpu.CompilerParams(use_tc_tiling_on_sc=False),
        scratch_types=[
            pltpu.VMEM((K, C), jnp.int32),
            pltpu.VMEM((K, C), jnp.int32),
            pltpu.VMEM((NB, C, W), jnp.float32),
            pltpu.VMEM_SHARED((N_PAD, W), jnp.float32),
            pltpu.VMEM_SHARED((N_PAD, W), jnp.float32),
            pltpu.SemaphoreType.DMA,
            pltpu.SemaphoreType.DMA,
        ],
    )
    def body(hw2_hbm, ei_hbm, zeros_hbm, out_hbm,
             src_v, dst_v, rows_v, table, agg, gsem, ssem):
        c = lax.axis_index("c")
        s = lax.axis_index("s")
        wid = s * NC + c
        pltpu.sync_copy(hw2_hbm.at[pl.ds(s * RPS, RPS)],
                        table.at[pl.ds(s * RPS, RPS)])
        pltpu.sync_copy(zeros_hbm.at[pl.ds(s * RPS, RPS)],
                        agg.at[pl.ds(s * RPS, RPS)])
        pltpu.sync_copy(ei_hbm.at[0, wid], src_v)
        pltpu.sync_copy(ei_hbm.at[1, wid], dst_v)
        plsc.subcore_barrier()

        def group(g, carry):
            p = (g % 2) * GP

            @pl.when(g >= 2)
            def _():
                for i in range(GP):
                    pltpu.make_async_copy(
                        rows_v.at[p + i], agg.at[pl.ds(0, C)], ssem).wait()

            gds = [
                pltpu.async_copy(
                    table.at[src_v.at[g * GP + i]], rows_v.at[p + i], gsem)
                for i in range(GP)
            ]
            for i in range(GP):
                gds[i].wait()
                pltpu.async_copy(
                    rows_v.at[p + i], agg.at[dst_v.at[g * GP + i]], ssem,
                    add=True)
            return carry

        lax.fori_loop(0, KG, group, 0)
        for i in range(2 * GP):
            pltpu.make_async_copy(
                rows_v.at[i % NB], agg.at[pl.ds(0, C)], ssem).wait()
        plsc.subcore_barrier()
        pltpu.sync_copy(agg.at[pl.ds(s * RPS, RPS)],
                        out_hbm.at[c, pl.ds(s * RPS, RPS)])

    return body(hw2, ei, zeros)


def _finish_body(hw2_ref, agg_ref, batch_ref, b2_ref, eps_ref, t_ref, o_ref):
    eps = eps_ref[0, 0]
    invt = 1.0 / t_ref[0, 0]
    z = (1.0 + eps) * hw2_ref[...] + agg_ref[0] + agg_ref[1] + b2_ref[...]
    col = lax.broadcasted_iota(jnp.int32, (N_PAD, W), 1)
    valid = col < D_OUT
    m = jnp.max(jnp.where(valid, z, -jnp.inf), axis=1, keepdims=True)
    e = jnp.where(valid, jnp.exp(z - m), 0.0)
    ls = z - m - jnp.log(jnp.sum(e, axis=1, keepdims=True))
    lsx = jnp.where(valid, ls, jnp.where(col == D_OUT, 1.0, 0.0))
    onehot_t = (batch_ref[...] ==
                lax.broadcasted_iota(jnp.int32, (G, N_PAD), 0))
    sums = lax.dot_general(onehot_t.astype(jnp.float32), lsx,
                           (((1,), (0,)), ((), ())),
                           preferred_element_type=jnp.float32)
    colg = lax.broadcasted_iota(jnp.int32, (G, W), 1)
    cnt = jnp.sum(jnp.where(colg == D_OUT, sums, 0.0), axis=1, keepdims=True)
    pooled = sums / jnp.maximum(cnt, 1.0) * invt
    validg = colg < D_OUT
    pm = jnp.max(jnp.where(validg, pooled, -jnp.inf), axis=1, keepdims=True)
    pe = jnp.where(validg, jnp.exp(pooled - pm), 0.0)
    o_ref[...] = pooled - pm - jnp.log(jnp.sum(pe, axis=1, keepdims=True))


def _finish(hw2, aggp, batch_row, b2p, eps, T):
    return pl.pallas_call(
        _finish_body,
        in_specs=[
            pl.BlockSpec((N_PAD, W), lambda: (0, 0)),
            pl.BlockSpec((NC, N_PAD, W), lambda: (0, 0, 0)),
            pl.BlockSpec((1, N_PAD), lambda: (0, 0)),
            pl.BlockSpec((1, W), lambda: (0, 0)),
            pl.BlockSpec(memory_space=pltpu.SMEM),
            pl.BlockSpec(memory_space=pltpu.SMEM),
        ],
        out_specs=pl.BlockSpec((G, W), lambda: (0, 0)),
        out_shape=jax.ShapeDtypeStruct((G, W), jnp.float32),
    )(hw2, aggp, batch_row, b2p, eps, T)


def kernel(x, edge_index, batch, W1, b1, W2, b2, eps, T):
    w2p = jnp.pad(W2, ((0, 0), (0, W - D_OUT)))
    b2p = jnp.pad(b2, (0, W - D_OUT)).reshape(1, W)
    b1r = b1.reshape(1, D_H)
    xp = jnp.pad(x, ((0, N_PAD - N), (0, 0)))
    hw2p = _mlp(xp, W1, b1r, w2p)
    npad_e = E_PAD - E
    pad_edges = jnp.stack([
        jnp.zeros((npad_e,), jnp.int32),
        N + (jnp.arange(npad_e, dtype=jnp.int32) % (N_PAD - N)),
    ])
    ei = jnp.concatenate([edge_index, pad_edges], axis=1).reshape(2, NW, K, C)
    zeros = jnp.zeros((N_PAD, W), jnp.float32)
    aggp = _sc_scatter(hw2p, ei, zeros)
    batch_row = jnp.concatenate(
        [batch, jnp.full((N_PAD - N,), G, jnp.int32)]).reshape(1, N_PAD)
    out16 = _finish(hw2p, aggp, batch_row, b2p,
                    eps.reshape(1, 1), T.reshape(1, 1))
    return out16[:, :D_OUT]


# speedup vs baseline: 28.1613x; 1.0116x over previous
import functools

import jax
import jax.numpy as jnp
from jax import lax
from jax.experimental import pallas as pl
from jax.experimental.pallas import tpu as pltpu
from jax.experimental.pallas import tpu_sc as plsc

N = 10000
E = 160000
D_IN = 256
D_H = 512
D_OUT = 3
G = 64
W = 8
NC, NS = 2, 16
NW = NC * NS
C = 128
K = 40
E_PAD = NW * K * C
N_PAD = 10240
RPS = N_PAD // NS
RBLK = 1024
NB = 8
GP = 4
KG = K // GP


def _mlp_body(x_ref, w1_ref, b1_ref, w2_ref, o_ref):
    h = jnp.dot(x_ref[...], w1_ref[...], preferred_element_type=jnp.float32)
    h = jnp.maximum(h + b1_ref[...], 0.0)
    o_ref[...] = jnp.dot(h, w2_ref[...], preferred_element_type=jnp.float32)


def _mlp(x, W1, b1r, w2p):
    return pl.pallas_call(
        _mlp_body,
        grid=(N_PAD // RBLK,),
        in_specs=[
            pl.BlockSpec((RBLK, D_IN), lambda i: (i, 0)),
            pl.BlockSpec((D_IN, D_H), lambda i: (0, 0)),
            pl.BlockSpec((1, D_H), lambda i: (0, 0)),
            pl.BlockSpec((D_H, W), lambda i: (0, 0)),
        ],
        out_specs=pl.BlockSpec((RBLK, W), lambda i: (i, 0)),
        out_shape=jax.ShapeDtypeStruct((N_PAD, W), jnp.float32),
    )(x, W1, b1r, w2p)


def _sc_scatter(hw2, ei, zeros):
    mesh = plsc.VectorSubcoreMesh(core_axis_name="c", subcore_axis_name="s")

    @functools.partial(
        pl.kernel,
        out_type=jax.ShapeDtypeStruct((NC, N_PAD, W), jnp.float32),
        mesh=mesh,
        compiler_params=pltpu.CompilerParams(use_tc_tiling_on_sc=False),
        scratch_types=[
            pltpu.VMEM((K, C), jnp.int32),
            pltpu.VMEM((K, C), jnp.int32),
            pltpu.VMEM((NB, C, W), jnp.float32),
            pltpu.VMEM_SHARED((N_PAD, W), jnp.float32),
            pltpu.VMEM_SHARED((N_PAD, W), jnp.float32),
            pltpu.SemaphoreType.DMA,
            pltpu.SemaphoreType.DMA,
        ],
    )
    def body(hw2_hbm, ei_hbm, zeros_hbm, out_hbm,
             src_v, dst_v, rows_v, table, agg, gsem, ssem):
        c = lax.axis_index("c")
        s = lax.axis_index("s")
        wid = s * NC + c
        pltpu.sync_copy(hw2_hbm.at[pl.ds(s * RPS, RPS)],
                        table.at[pl.ds(s * RPS, RPS)])
        pltpu.sync_copy(zeros_hbm.at[pl.ds(s * RPS, RPS)],
                        agg.at[pl.ds(s * RPS, RPS)])
        pltpu.sync_copy(ei_hbm.at[0, wid], src_v)
        pltpu.sync_copy(ei_hbm.at[1, wid], dst_v)
        plsc.subcore_barrier()

        def group(g, carry):
            p = (g % 2) * GP

            @pl.when(g >= 2)
            def _():
                for i in range(GP):
                    pltpu.make_async_copy(
                        rows_v.at[p + i], agg.at[pl.ds(0, C)], ssem).wait()

            gds = [
                pltpu.async_copy(
                    table.at[src_v.at[g * GP + i]], rows_v.at[p + i], gsem)
                for i in range(GP)
            ]
            for i in range(GP):
                gds[i].wait()
                pltpu.async_copy(
                    rows_v.at[p + i], agg.at[dst_v.at[g * GP + i]], ssem,
                    add=True)
            return carry

        lax.fori_loop(0, KG, group, 0)
        for i in range(2 * GP):
            pltpu.make_async_copy(
                rows_v.at[i % NB], agg.at[pl.ds(0, C)], ssem).wait()
        plsc.subcore_barrier()
        pltpu.sync_copy(agg.at[pl.ds(s * RPS, RPS)],
                        out_hbm.at[c, pl.ds(s * RPS, RPS)])

    return body(hw2, ei, zeros)


def _finish_body(hw2_ref, agg_ref, batch_ref, b2_ref, eps_ref, t_ref, o_ref):
    eps = eps_ref[0, 0]
    invt = 1.0 / t_ref[0, 0]
    z = (1.0 + eps) * hw2_ref[...] + agg_ref[0] + agg_ref[1] + b2_ref[...]
    col = lax.broadcasted_iota(jnp.int32, (N_PAD, W), 1)
    valid = col < D_OUT
    m = jnp.max(jnp.where(valid, z, -jnp.inf), axis=1, keepdims=True)
    e = jnp.where(valid, jnp.exp(z - m), 0.0)
    ls = z - m - jnp.log(jnp.sum(e, axis=1, keepdims=True))
    lsx = jnp.where(valid, ls, jnp.where(col == D_OUT, 1.0, 0.0))
    onehot_t = (batch_ref[...] ==
                lax.broadcasted_iota(jnp.int32, (G, N_PAD), 0))
    sums = lax.dot_general(onehot_t.astype(jnp.float32), lsx,
                           (((1,), (0,)), ((), ())),
                           preferred_element_type=jnp.float32)
    colg = lax.broadcasted_iota(jnp.int32, (G, W), 1)
    cnt = jnp.sum(jnp.where(colg == D_OUT, sums, 0.0), axis=1, keepdims=True)
    pooled = sums / jnp.maximum(cnt, 1.0) * invt
    validg = colg < D_OUT
    pm = jnp.max(jnp.where(validg, pooled, -jnp.inf), axis=1, keepdims=True)
    pe = jnp.where(validg, jnp.exp(pooled - pm), 0.0)
    o_ref[...] = pooled - pm - jnp.log(jnp.sum(pe, axis=1, keepdims=True))


def _finish(hw2, aggp, batch_row, b2p, eps, T):
    return pl.pallas_call(
        _finish_body,
        in_specs=[
            pl.BlockSpec((N_PAD, W), lambda: (0, 0)),
            pl.BlockSpec((NC, N_PAD, W), lambda: (0, 0, 0)),
            pl.BlockSpec((1, N_PAD), lambda: (0, 0)),
            pl.BlockSpec((1, W), lambda: (0, 0)),
            pl.BlockSpec(memory_space=pltpu.SMEM),
            pl.BlockSpec(memory_space=pltpu.SMEM),
        ],
        out_specs=pl.BlockSpec((G, W), lambda: (0, 0)),
        out_shape=jax.ShapeDtypeStruct((G, W), jnp.float32),
    )(hw2, aggp, batch_row, b2p, eps, T)


def kernel(x, edge_index, batch, W1, b1, W2, b2, eps, T):
    w2p = jnp.pad(W2, ((0, 0), (0, W - D_OUT)))
    b2p = jnp.pad(b2, (0, W - D_OUT)).reshape(1, W)
    b1r = b1.reshape(1, D_H)
    xp = jnp.pad(x.astype(jnp.bfloat16), ((0, N_PAD - N), (0, 0)))
    hw2p = _mlp(xp, W1.astype(jnp.bfloat16), b1r, w2p)
    npad_e = E_PAD - E
    pad_edges = jnp.stack([
        jnp.zeros((npad_e,), jnp.int32),
        N + (jnp.arange(npad_e, dtype=jnp.int32) % (N_PAD - N)),
    ])
    ei = jnp.concatenate([edge_index, pad_edges], axis=1).reshape(2, NW, K, C)
    zeros = jnp.zeros((N_PAD, W), jnp.float32)
    aggp = _sc_scatter(hw2p, ei, zeros)
    batch_row = jnp.concatenate(
        [batch, jnp.full((N_PAD - N,), G, jnp.int32)]).reshape(1, N_PAD)
    out16 = _finish(hw2p, aggp, batch_row, b2p,
                    eps.reshape(1, 1), T.reshape(1, 1))
    return out16[:, :D_OUT]


# speedup vs baseline: 41.3638x; 1.4688x over previous
import functools

import jax
import jax.numpy as jnp
from jax import lax
from jax.experimental import pallas as pl
from jax.experimental.pallas import tpu as pltpu
from jax.experimental.pallas import tpu_sc as plsc

N = 10000
E = 160000
D_IN = 256
D_H = 512
D_OUT = 3
G = 64
W = 8
NC, NS = 2, 16
NW = NC * NS
C = 128
K = 40
E_PAD = NW * K * C
N_PAD = 10240
RPS = N_PAD // NS
RBLK = 1000
NB = 8
GP = 4
KG = K // GP
PKL = 128
PKR = N_PAD * W // PKL


def _mlp_body(x_ref, w1_ref, b1_ref, w2_ref, o_ref):
    h = jnp.dot(x_ref[...], w1_ref[...], preferred_element_type=jnp.float32)
    h = jnp.maximum(h + b1_ref[...], 0.0).astype(jnp.bfloat16)
    o_ref[...] = jnp.dot(h, w2_ref[...], preferred_element_type=jnp.float32)


def _mlp(x, W1, b1r, w2p):
    return pl.pallas_call(
        _mlp_body,
        grid=(N // RBLK,),
        in_specs=[
            pl.BlockSpec((RBLK, D_IN), lambda i: (i, 0)),
            pl.BlockSpec((D_IN, D_H), lambda i: (0, 0)),
            pl.BlockSpec((1, D_H), lambda i: (0, 0)),
            pl.BlockSpec((D_H, W), lambda i: (0, 0)),
        ],
        out_specs=pl.BlockSpec((RBLK, W), lambda i: (i, 0)),
        out_shape=jax.ShapeDtypeStruct((N_PAD, W), jnp.float32),
    )(x, W1, b1r, w2p)


def _sc_scatter(hw2, ei, zeros):
    mesh = plsc.VectorSubcoreMesh(core_axis_name="c", subcore_axis_name="s")

    @functools.partial(
        pl.kernel,
        out_type=jax.ShapeDtypeStruct((NC, N_PAD, W), jnp.float32),
        mesh=mesh,
        compiler_params=pltpu.CompilerParams(use_tc_tiling_on_sc=False),
        scratch_types=[
            pltpu.VMEM((K, C), jnp.int32),
            pltpu.VMEM((K, C), jnp.int32),
            pltpu.VMEM((NB, C, W), jnp.float32),
            pltpu.VMEM_SHARED((N_PAD, W), jnp.float32),
            pltpu.VMEM_SHARED((N_PAD, W), jnp.float32),
            pltpu.SemaphoreType.DMA,
            pltpu.SemaphoreType.DMA,
        ],
    )
    def body(hw2_hbm, ei_hbm, zeros_hbm, out_hbm,
             src_v, dst_v, rows_v, table, agg, gsem, ssem):
        c = lax.axis_index("c")
        s = lax.axis_index("s")
        wid = s * NC + c
        pltpu.sync_copy(hw2_hbm.at[pl.ds(s * RPS, RPS)],
                        table.at[pl.ds(s * RPS, RPS)])
        pltpu.sync_copy(zeros_hbm.at[pl.ds(s * RPS, RPS)],
                        agg.at[pl.ds(s * RPS, RPS)])
        pltpu.sync_copy(ei_hbm.at[0, wid], src_v)
        pltpu.sync_copy(ei_hbm.at[1, wid], dst_v)
        plsc.subcore_barrier()

        def group(g, carry):
            p = (g % 2) * GP

            @pl.when(g >= 2)
            def _():
                for i in range(GP):
                    pltpu.make_async_copy(
                        rows_v.at[p + i], agg.at[pl.ds(0, C)], ssem).wait()

            gds = [
                pltpu.async_copy(
                    table.at[src_v.at[g * GP + i]], rows_v.at[p + i], gsem)
                for i in range(GP)
            ]
            for i in range(GP):
                gds[i].wait()
                pltpu.async_copy(
                    rows_v.at[p + i], agg.at[dst_v.at[g * GP + i]], ssem,
                    add=True)
            return carry

        lax.fori_loop(0, KG, group, 0)
        for i in range(2 * GP):
            pltpu.make_async_copy(
                rows_v.at[i % NB], agg.at[pl.ds(0, C)], ssem).wait()
        plsc.subcore_barrier()
        pltpu.sync_copy(agg.at[pl.ds(s * RPS, RPS)],
                        out_hbm.at[c, pl.ds(s * RPS, RPS)])

    return body(hw2, ei, zeros)


def _finish_body(hw2_ref, agg_ref, batch_ref, b2_ref, eps_ref, t_ref, o_ref):
    eps = eps_ref[0, 0]
    invt = 1.0 / t_ref[0, 0]
    zp = ((1.0 + eps) * hw2_ref[...] + agg_ref[0] + agg_ref[1] +
          b2_ref[...])
    z = jnp.concatenate([zp[:, 8 * j:8 * j + 8] for j in range(16)], axis=0)
    row = lax.broadcasted_iota(jnp.int32, (N_PAD, W), 0)
    z = jnp.where(lax.rem(row, PKR) < N // 16, z, 0.0)
    col = lax.broadcasted_iota(jnp.int32, (N_PAD, W), 1)
    valid = col < D_OUT
    m = jnp.max(jnp.where(valid, z, -jnp.inf), axis=1, keepdims=True)
    e = jnp.where(valid, jnp.exp(z - m), 0.0)
    ls = z - m - jnp.log(jnp.sum(e, axis=1, keepdims=True))
    lsx = jnp.where(valid, ls, jnp.where(col == D_OUT, 1.0, 0.0))
    onehot_t = (batch_ref[...] ==
                lax.broadcasted_iota(jnp.int32, (G, N_PAD), 0))
    sums = lax.dot_general(onehot_t.astype(jnp.bfloat16),
                           lsx.astype(jnp.bfloat16),
                           (((1,), (0,)), ((), ())),
                           preferred_element_type=jnp.float32)
    colg = lax.broadcasted_iota(jnp.int32, (G, W), 1)
    cnt = jnp.sum(jnp.where(colg == D_OUT, sums, 0.0), axis=1, keepdims=True)
    pooled = sums / jnp.maximum(cnt, 1.0) * invt
    validg = colg < D_OUT
    pm = jnp.max(jnp.where(validg, pooled, -jnp.inf), axis=1, keepdims=True)
    pe = jnp.where(validg, jnp.exp(pooled - pm), 0.0)
    o_ref[...] = pooled - pm - jnp.log(jnp.sum(pe, axis=1, keepdims=True))


def _finish(hw2pk, aggpk, batch_row, b2t, eps, T):
    return pl.pallas_call(
        _finish_body,
        in_specs=[
            pl.BlockSpec((PKR, PKL), lambda: (0, 0)),
            pl.BlockSpec((NC, PKR, PKL), lambda: (0, 0, 0)),
            pl.BlockSpec((1, N_PAD), lambda: (0, 0)),
            pl.BlockSpec((1, PKL), lambda: (0, 0)),
            pl.BlockSpec(memory_space=pltpu.SMEM),
            pl.BlockSpec(memory_space=pltpu.SMEM),
        ],
        out_specs=pl.BlockSpec((G, W), lambda: (0, 0)),
        out_shape=jax.ShapeDtypeStruct((G, W), jnp.float32),
    )(hw2pk, aggpk, batch_row, b2t, eps, T)


def kernel(x, edge_index, batch, W1, b1, W2, b2, eps, T):
    w2p = jnp.pad(W2, ((0, 0), (0, W - D_OUT))).astype(jnp.bfloat16)
    b2t = jnp.tile(jnp.pad(b2, (0, W - D_OUT)), PKL // W).reshape(1, PKL)
    b1r = b1.reshape(1, D_H)
    hw2 = _mlp(x.astype(jnp.bfloat16), W1.astype(jnp.bfloat16), b1r, w2p)
    hw2f = hw2.reshape(N_PAD * W)
    npad_e = E_PAD - E
    pad_edges = jnp.stack([
        jnp.zeros((npad_e,), jnp.int32),
        N + (jnp.arange(npad_e, dtype=jnp.int32) % (N_PAD - N)),
    ])
    ei = jnp.concatenate([edge_index, pad_edges], axis=1).reshape(2, NW, K, C)
    zeros = jnp.zeros((N_PAD, W), jnp.float32)
    aggp = _sc_scatter(hw2f.reshape(N_PAD, W), ei, zeros)
    aggpk = aggp.reshape(NC, PKR, PKL)
    batch_row = jnp.concatenate(
        [batch, jnp.full((N_PAD - N,), G, jnp.int32)]
    ).reshape(PKR, 16).T.reshape(1, N_PAD)
    out16 = _finish(hw2f.reshape(PKR, PKL), aggpk, batch_row, b2t,
                    eps.reshape(1, 1), T.reshape(1, 1))
    return out16[:, :D_OUT]
